# baseline (speedup 1.0000x reference)
import functools

import jax
import jax.numpy as jnp
from jax import lax
from jax.experimental import pallas as pl
from jax.experimental.pallas import tpu as pltpu
from jax.experimental.pallas import tpu_sc as plsc

B = 16384
V = 1000000
D = 32
H = 256

NC = 2
NS = 16
NW = NC * NS
BPW = B // NW
CH = 128
NCH = BPW // CH


def _sc_gather(user2d, item2d, ubt, ibt, uet, iet):
    mesh = plsc.VectorSubcoreMesh(core_axis_name="c", subcore_axis_name="s")

    @functools.partial(
        pl.kernel,
        out_type=(
            jax.ShapeDtypeStruct((B, D), jnp.float32),
            jax.ShapeDtypeStruct((B, D), jnp.float32),
            jax.ShapeDtypeStruct((B,), jnp.float32),
        ),
        mesh=mesh,
        scratch_types=[
            pltpu.VMEM((NCH, CH), jnp.int32),
            pltpu.VMEM((NCH, CH), jnp.int32),
            pltpu.VMEM((BPW, D), jnp.float32),
            pltpu.VMEM((BPW, D), jnp.float32),
            pltpu.VMEM((BPW,), jnp.float32),
            pltpu.VMEM((BPW,), jnp.float32),
            pltpu.SemaphoreType.DMA,
        ],
        compiler_params=pltpu.CompilerParams(use_tc_tiling_on_sc=False),
    )
    def k(user_h, item_h, ubt_h, ibt_h, uet_h, iet_h, xu_h, xi_h, wide_h,
          idx_u, idx_i, rows_u, rows_i, bu, bi, sem):
        wid = lax.axis_index("s") * NC + lax.axis_index("c")
        base = wid * BPW
        pltpu.sync_copy(user_h.at[pl.ds(wid * NCH, NCH)], idx_u)
        pltpu.sync_copy(item_h.at[pl.ds(wid * NCH, NCH)], idx_i)
        copies = []
        for j in range(NCH):
            dst = pl.ds(j * CH, CH)
            copies.append(pltpu.async_copy(uet_h.at[idx_u.at[j]],
                                           rows_u.at[dst], sem))
            copies.append(pltpu.async_copy(iet_h.at[idx_i.at[j]],
                                           rows_i.at[dst], sem))
            copies.append(pltpu.async_copy(ubt_h.at[idx_u.at[j]],
                                           bu.at[dst], sem))
            copies.append(pltpu.async_copy(ibt_h.at[idx_i.at[j]],
                                           bi.at[dst], sem))
        for c in copies:
            c.wait()
        for i in range(BPW // 16):
            s = pl.ds(i * 16, 16)
            bu[s] = bu[s] + bi[s]
        pltpu.sync_copy(rows_u, xu_h.at[pl.ds(base, BPW)])
        pltpu.sync_copy(rows_i, xi_h.at[pl.ds(base, BPW)])
        pltpu.sync_copy(bu, wide_h.at[pl.ds(base, BPW)])

    return k(user2d, item2d, ubt, ibt, uet, iet)


BT = 1024


def _mlp_body(xu_r, xi_r, w_r, W1u_r, W1i_r, b1_r, W2_r, b2_r, W3_r, b3_r,
              out_r):
    h = jnp.dot(xu_r[:], W1u_r[:], preferred_element_type=jnp.float32)
    h = h + jnp.dot(xi_r[:], W1i_r[:], preferred_element_type=jnp.float32)
    h = jnp.maximum(h + b1_r[:], 0.0)
    h = jnp.maximum(
        jnp.dot(h, W2_r[:], preferred_element_type=jnp.float32) + b2_r[:], 0.0)
    out_r[:] = (jnp.dot(h, W3_r[:], preferred_element_type=jnp.float32)
                + w_r[:] + b3_r[:])


def _mlp(xu, xi, wide2d, W1u, W1i, b1, W2, b2, W3, b3):
    rep = lambda shape: pl.BlockSpec(shape, lambda i: tuple(0 for _ in shape))
    return pl.pallas_call(
        _mlp_body,
        grid=(B // BT,),
        in_specs=[
            pl.BlockSpec((BT, D), lambda i: (i, 0)),
            pl.BlockSpec((BT, D), lambda i: (i, 0)),
            pl.BlockSpec((BT, 1), lambda i: (i, 0)),
            rep((D, H)),
            rep((D, H)),
            rep((H,)),
            rep((H, H // 2)),
            rep((H // 2,)),
            rep((H // 2, 1)),
            rep((1,)),
        ],
        out_specs=pl.BlockSpec((BT, 1), lambda i: (i, 0)),
        out_shape=jax.ShapeDtypeStruct((B, 1), jnp.float32),
    )(xu, xi, wide2d, W1u, W1i, b1, W2, b2, W3, b3)


def kernel(user, item, user_bias_table, item_bias_table, user_emb_table,
           item_emb_table, W1, b1, W2, b2, W3, b3):
    user2d = user.astype(jnp.int32).reshape(B // CH, CH)
    item2d = item.astype(jnp.int32).reshape(B // CH, CH)
    ubt = user_bias_table.reshape(V)
    ibt = item_bias_table.reshape(V)
    xu, xi, wide = _sc_gather(user2d, item2d, ubt, ibt,
                              user_emb_table, item_emb_table)
    out = _mlp(xu, xi, wide.reshape(B, 1), W1[:D], W1[D:], b1, W2, b2, W3, b3)
    return out.reshape(B)
